# Initial kernel scaffold; baseline (speedup 1.0000x reference)
import jax
import jax.numpy as jnp
from jax.experimental import pallas as pl


def kernel(user_index, item_index, edge_index, edge_vals, user_emb, item_emb, W1_0, W2_0, W1_1, W2_1):
    raise NotImplementedError("write your pallas kernel here")


# speedup vs baseline: 2.0334x; 2.0334x over previous
import functools

import jax
import jax.numpy as jnp
from jax import lax
from jax.experimental import pallas as pl
from jax.experimental.pallas import tpu as pltpu
from jax.experimental.pallas import tpu_sc as plsc

NUM_USERS = 25000
NUM_ITEMS = 25000
N = NUM_USERS + NUM_ITEMS
E = 800000
D = 64
B = 4096

NC = 2
NS = 16
HALF = N // NC
ACC_ROWS = 25600
EPT = E // NS
CH = 80
NCHUNK = EPT // CH
ZR = 200
OUT_PT = 1560

_mesh = plsc.VectorSubcoreMesh(core_axis_name="c", subcore_axis_name="s")
_sc_params = pltpu.CompilerParams(use_tc_tiling_on_sc=False,
                                  needs_layout_passes=False)


def _spmm_body(dst_hbm, src_hbm, val_hbm, x_hbm, rel_hbm,
               dst_v, src_v, val_v, dstl_v, rows_v, stage_v, acc, sem):
    c = lax.axis_index("c")
    s = lax.axis_index("s")
    iota = lax.iota(jnp.int32, 16)

    def _zb(i, _):
        for q in range(4):
            stage_v[i, pl.ds(q * 16, 16)] = jnp.zeros((16,), jnp.float32)
        return 0
    lax.fori_loop(0, ZR, _zb, 0)
    for j in range(ACC_ROWS // NS // ZR):
        pltpu.sync_copy(stage_v, acc.at[pl.ds(s * (ACC_ROWS // NS) + j * ZR, ZR)])
    plsc.subcore_barrier()

    half_base = c * HALF

    def _chunk(k, _):
        base = s * EPT + k * CH
        pltpu.sync_copy(dst_hbm.at[pl.ds(base, CH)], dst_v)
        pltpu.sync_copy(src_hbm.at[pl.ds(base, CH)], src_v)
        pltpu.sync_copy(val_hbm.at[pl.ds(base, CH)], val_v)
        for g in range(CH // 16):
            d = dst_v[pl.ds(g * 16, 16)] - half_base
            m = (d >= 0) & (d < HALF)
            junk = HALF + ((k * CH + g * 16) % 512) + iota
            dstl_v[pl.ds(g * 16, 16)] = jnp.where(m, d, junk)
        pltpu.async_copy(x_hbm.at[src_v], rows_v, sem).wait()

        def _scale(g, _):
            val16 = val_v[pl.ds(g * 16, 16)]
            for e in range(16):
                v = val16[e]
                r = g * 16 + e
                for q in range(4):
                    qs = pl.ds(q * 16, 16)
                    rows_v[r, qs] = rows_v[r, qs] * v
            return 0
        lax.fori_loop(0, CH // 16, _scale, 0)
        pltpu.sync_copy(rows_v, acc.at[dstl_v], add=True)
        return 0
    lax.fori_loop(0, NCHUNK, _chunk, 0)
    plsc.subcore_barrier()

    for j in range(8):
        lo = s * OUT_PT + j * ZR
        nr = ZR if j < 7 else OUT_PT - 7 * ZR
        pltpu.sync_copy(acc.at[pl.ds(lo, nr)], stage_v.at[pl.ds(0, nr)])
        pltpu.sync_copy(stage_v.at[pl.ds(0, nr)],
                        rel_hbm.at[pl.ds(half_base + lo, nr)])

    @pl.when(s == NS - 1)
    def _tail():
        lo = NS * OUT_PT
        pltpu.sync_copy(acc.at[pl.ds(lo, HALF - lo)],
                        stage_v.at[pl.ds(0, HALF - lo)])
        pltpu.sync_copy(stage_v.at[pl.ds(0, HALF - lo)],
                        rel_hbm.at[pl.ds(half_base + lo, HALF - lo)])


_spmm = functools.partial(
    pl.kernel,
    out_type=jax.ShapeDtypeStruct((N, D), jnp.float32),
    mesh=_mesh,
    scratch_types=[
        pltpu.VMEM((CH,), jnp.int32),
        pltpu.VMEM((CH,), jnp.int32),
        pltpu.VMEM((CH,), jnp.float32),
        pltpu.VMEM((CH,), jnp.int32),
        pltpu.VMEM((CH, D), jnp.float32),
        pltpu.VMEM((ZR, D), jnp.float32),
        pltpu.VMEM_SHARED((ACC_ROWS, D), jnp.float32),
        pltpu.SemaphoreType.DMA,
    ],
    compiler_params=_sc_params,
)(_spmm_body)


PB = B // (NC * NS)


def _score_body(ui_hbm, ii_hbm, uemb_hbm, iemb_hbm, out1_hbm, out2_hbm,
                scores_hbm, ui_v, ii_v, ue_v, ie_v, p1u_v, p1i_v, p2u_v,
                p2i_v, sc_v, sem):
    c = lax.axis_index("c")
    s = lax.axis_index("s")
    wid = s * NC + c
    base = wid * PB
    iota = lax.iota(jnp.int32, 16)

    pltpu.sync_copy(ui_hbm.at[pl.ds(base, PB)], ui_v)
    pltpu.sync_copy(ii_hbm.at[pl.ds(base, PB)], ii_v)
    pltpu.async_copy(uemb_hbm.at[ui_v], ue_v, sem).wait()
    pltpu.async_copy(out1_hbm.at[ui_v], p1u_v, sem).wait()
    pltpu.async_copy(out2_hbm.at[ui_v], p2u_v, sem).wait()
    pltpu.async_copy(iemb_hbm.at[ii_v], ie_v, sem).wait()
    for g in range(PB // 16):
        ii_v[pl.ds(g * 16, 16)] = ii_v[pl.ds(g * 16, 16)] + NUM_USERS
    pltpu.async_copy(out1_hbm.at[ii_v], p1i_v, sem).wait()
    pltpu.async_copy(out2_hbm.at[ii_v], p2i_v, sem).wait()

    def _pair(e, _):
        acc = jnp.zeros((16,), jnp.float32)
        for q in range(4):
            qs = pl.ds(q * 16, 16)
            acc = acc + ue_v[e, qs] * ie_v[e, qs]
            acc = acc + p1u_v[e, qs] * p1i_v[e, qs]
            acc = acc + p2u_v[e, qs] * p2i_v[e, qs]
        tot = jnp.sum(acc)
        plsc.store_scatter(sc_v, [jnp.full((16,), e, jnp.int32)],
                           jnp.full((16,), 0.0, jnp.float32) + tot,
                           mask=iota == 0)
        return 0
    lax.fori_loop(0, PB, _pair, 0)
    pltpu.sync_copy(sc_v, scores_hbm.at[pl.ds(base, PB)])


_score = functools.partial(
    pl.kernel,
    out_type=jax.ShapeDtypeStruct((B,), jnp.float32),
    mesh=_mesh,
    scratch_types=[
        pltpu.VMEM((PB,), jnp.int32),
        pltpu.VMEM((PB,), jnp.int32),
        pltpu.VMEM((PB, D), jnp.float32),
        pltpu.VMEM((PB, D), jnp.float32),
        pltpu.VMEM((PB, D), jnp.float32),
        pltpu.VMEM((PB, D), jnp.float32),
        pltpu.VMEM((PB, D), jnp.float32),
        pltpu.VMEM((PB, D), jnp.float32),
        pltpu.VMEM((PB,), jnp.float32),
        pltpu.SemaphoreType.DMA,
    ],
    compiler_params=_sc_params,
)(_score_body)


BR = 2000


def _dense_body(x_ref, rel_ref, w1_ref, w2_ref, o_ref):
    x = x_ref[...]
    rel = rel_ref[...]
    a = jnp.dot(x + rel, w1_ref[...], preferred_element_type=jnp.float32)
    b = jnp.dot(rel * x, w2_ref[...], preferred_element_type=jnp.float32)
    o = a + b
    o_ref[...] = jnp.where(o >= 0, o, 0.2 * o)


def _dense(x, rel, w1, w2):
    return pl.pallas_call(
        _dense_body,
        grid=(N // BR,),
        in_specs=[
            pl.BlockSpec((BR, D), lambda i: (i, 0)),
            pl.BlockSpec((BR, D), lambda i: (i, 0)),
            pl.BlockSpec((D, D), lambda i: (0, 0)),
            pl.BlockSpec((D, D), lambda i: (0, 0)),
        ],
        out_specs=pl.BlockSpec((BR, D), lambda i: (i, 0)),
        out_shape=jax.ShapeDtypeStruct((N, D), jnp.float32),
    )(x, rel, w1, w2)


def kernel(user_index, item_index, edge_index, edge_vals, user_emb, item_emb,
           W1_0, W2_0, W1_1, W2_1):
    x = jnp.concatenate([user_emb, item_emb], axis=0)
    dst = edge_index[0]
    src = edge_index[1]
    rel1 = _spmm(dst, src, edge_vals, x)
    out1 = _dense(x, rel1, W1_0, W2_0)
    rel2 = _spmm(dst, src, edge_vals, out1)
    out2 = _dense(out1, rel2, W1_1, W2_1)
    return _score(user_index, item_index, user_emb, item_emb, out1, out2)


# speedup vs baseline: 3.2822x; 1.6142x over previous
import functools

import jax
import jax.numpy as jnp
from jax import lax
from jax.experimental import pallas as pl
from jax.experimental.pallas import tpu as pltpu
from jax.experimental.pallas import tpu_sc as plsc

NUM_USERS = 25000
NUM_ITEMS = 25000
N = NUM_USERS + NUM_ITEMS
E = 800000
D = 64
B = 4096

NC = 2
NS = 16
HALF = N // NC
ACC_ROWS = 25600
EPT = E // NS
CH = 80
NCHUNK = EPT // CH
BE = 2000
CPB = BE // CH
NBLK = EPT // BE
ZPT = ACC_ROWS // NS
OUT_PT = 1560

_mesh = plsc.VectorSubcoreMesh(core_axis_name="c", subcore_axis_name="s")
_sc_params = pltpu.CompilerParams(use_tc_tiling_on_sc=False,
                                  needs_layout_passes=False)


def _spmm_body(dst_hbm, src_hbm, val_hbm, x_hbm, rel_hbm,
               dstb, srcb, valb,
               dstl0, dstl1, dstl2, srcc0, srcc1, srcc2,
               rows0, rows1, rows2, acc,
               esem, gsem0, gsem1, gsem2, ssem0, ssem1, ssem2):
    c_ax = lax.axis_index("c")
    s = lax.axis_index("s")
    iota = lax.iota(jnp.int32, 16)
    half_base = c_ax * HALF
    dstl = (dstl0, dstl1, dstl2)
    srcc = (srcc0, srcc1, srcc2)
    rows = (rows0, rows1, rows2)
    gsem = (gsem0, gsem1, gsem2)
    ssem = (ssem0, ssem1, ssem2)

    def _zr(i, _):
        for q in range(4):
            rows0[i, pl.ds(q * 16, 16)] = jnp.zeros((16,), jnp.float32)
        return 0
    lax.fori_loop(0, CH, _zr, 0)

    def _zc(i, _):
        pltpu.sync_copy(rows0, acc.at[pl.ds(s * ZPT + i * CH, CH)])
        return 0
    lax.fori_loop(0, ZPT // CH, _zc, 0)
    plsc.subcore_barrier()

    def load_block(b):
        pb = b % 2
        tb = s * EPT + b * BE
        pltpu.async_copy(dst_hbm.at[pl.ds(tb, BE)], dstb.at[pb], esem)
        pltpu.async_copy(src_hbm.at[pl.ds(tb, BE)], srcb.at[pb], esem)
        pltpu.async_copy(val_hbm.at[pl.ds(tb, BE)], valb.at[pb], esem)

    def wait_block():
        pltpu.make_async_copy(dst_hbm.at[pl.ds(0, BE)], dstb.at[0], esem).wait()
        pltpu.make_async_copy(src_hbm.at[pl.ds(0, BE)], srcb.at[0], esem).wait()
        pltpu.make_async_copy(val_hbm.at[pl.ds(0, BE)], valb.at[0], esem).wait()

    def wait_scatter(i):
        pltpu.make_async_copy(rows[i], acc.at[dstl[i]], ssem[i]).wait()

    def prep_gather(c, i):
        @pl.when(c >= 3)
        def _():
            wait_scatter(i)
        b = c // CPB

        @pl.when((c % CPB == 0) & (c > 0))
        def _():
            wait_block()

        @pl.when((c % CPB == 2) & (b >= 1) & (b + 1 < NBLK))
        def _():
            load_block(b + 1)

        pb = b % 2
        off = (c % CPB) * CH
        for g in range(CH // 16):
            sl = pl.ds(off + g * 16, 16)
            d = dstb[pb, sl] - half_base
            m = (d >= 0) & (d < HALF)
            junk = HALF + (c * CH + g * 16) % 512 + iota
            dstl[i][pl.ds(g * 16, 16)] = jnp.where(m, d, junk)
            srcc[i][pl.ds(g * 16, 16)] = srcb[pb, sl]
        pltpu.async_copy(x_hbm.at[srcc[i]], rows[i], gsem[i])

    def process(c, i):
        pltpu.make_async_copy(x_hbm.at[srcc[i]], rows[i], gsem[i]).wait()
        pb = (c // CPB) % 2
        off = (c % CPB) * CH

        def _scale(g, _):
            val16 = valb[pb, pl.ds(off + g * 16, 16)]
            for e in range(16):
                v = val16[e]
                r = g * 16 + e
                for q in range(4):
                    qs = pl.ds(q * 16, 16)
                    rows[i][r, qs] = rows[i][r, qs] * v
            return 0
        lax.fori_loop(0, CH // 16, _scale, 0)
        pltpu.async_copy(rows[i], acc.at[dstl[i]], ssem[i], add=True)

    load_block(0)
    wait_block()
    load_block(1)
    prep_gather(jnp.int32(0), 0)
    prep_gather(jnp.int32(1), 1)

    def _jbody(j, _):
        for k in range(3):
            c = 3 * j + k
            cg = c + 2

            @pl.when(cg <= NCHUNK - 1)
            def _(cg=cg, k=k):
                prep_gather(cg, (2 + k) % 3)
            process(c, k)
        return 0
    lax.fori_loop(0, (NCHUNK - 1) // 3, _jbody, 0)

    process(jnp.int32(NCHUNK - 1), (NCHUNK - 1) % 3)
    for i in range(3):
        wait_scatter(i)
    plsc.subcore_barrier()

    def _wb(i, _):
        lo = s * OUT_PT + i * CH
        pltpu.sync_copy(acc.at[pl.ds(lo, CH)], rows0)
        pltpu.sync_copy(rows0, rel_hbm.at[pl.ds(half_base + lo, CH)])
        return 0
    lax.fori_loop(0, OUT_PT // CH, _wb, 0)
    lo40 = s * OUT_PT + (OUT_PT // CH) * CH
    pltpu.sync_copy(acc.at[pl.ds(lo40, OUT_PT - (OUT_PT // CH) * CH)],
                    rows0.at[pl.ds(0, OUT_PT - (OUT_PT // CH) * CH)])
    pltpu.sync_copy(rows0.at[pl.ds(0, OUT_PT - (OUT_PT // CH) * CH)],
                    rel_hbm.at[pl.ds(half_base + lo40,
                                     OUT_PT - (OUT_PT // CH) * CH)])

    @pl.when(s == NS - 1)
    def _tail():
        lo = NS * OUT_PT
        pltpu.sync_copy(acc.at[pl.ds(lo, HALF - lo)],
                        rows0.at[pl.ds(0, HALF - lo)])
        pltpu.sync_copy(rows0.at[pl.ds(0, HALF - lo)],
                        rel_hbm.at[pl.ds(half_base + lo, HALF - lo)])


_spmm = functools.partial(
    pl.kernel,
    out_type=jax.ShapeDtypeStruct((N, D), jnp.float32),
    mesh=_mesh,
    scratch_types=[
        pltpu.VMEM((2, BE), jnp.int32),
        pltpu.VMEM((2, BE), jnp.int32),
        pltpu.VMEM((2, BE), jnp.float32),
        pltpu.VMEM((CH,), jnp.int32),
        pltpu.VMEM((CH,), jnp.int32),
        pltpu.VMEM((CH,), jnp.int32),
        pltpu.VMEM((CH,), jnp.int32),
        pltpu.VMEM((CH,), jnp.int32),
        pltpu.VMEM((CH,), jnp.int32),
        pltpu.VMEM((CH, D), jnp.float32),
        pltpu.VMEM((CH, D), jnp.float32),
        pltpu.VMEM((CH, D), jnp.float32),
        pltpu.VMEM_SHARED((ACC_ROWS, D), jnp.float32),
        pltpu.SemaphoreType.DMA,
        pltpu.SemaphoreType.DMA,
        pltpu.SemaphoreType.DMA,
        pltpu.SemaphoreType.DMA,
        pltpu.SemaphoreType.DMA,
        pltpu.SemaphoreType.DMA,
        pltpu.SemaphoreType.DMA,
    ],
    compiler_params=_sc_params,
)(_spmm_body)


PB = B // (NC * NS)


def _score_body(ui_hbm, ii_hbm, uemb_hbm, iemb_hbm, out1_hbm, out2_hbm,
                scores_hbm, ui_v, ii_v, ue_v, ie_v, p1u_v, p1i_v, p2u_v,
                p2i_v, sc_v, sem):
    c = lax.axis_index("c")
    s = lax.axis_index("s")
    wid = s * NC + c
    base = wid * PB
    iota = lax.iota(jnp.int32, 16)

    pltpu.sync_copy(ui_hbm.at[pl.ds(base, PB)], ui_v)
    pltpu.sync_copy(ii_hbm.at[pl.ds(base, PB)], ii_v)
    pltpu.async_copy(uemb_hbm.at[ui_v], ue_v, sem).wait()
    pltpu.async_copy(out1_hbm.at[ui_v], p1u_v, sem).wait()
    pltpu.async_copy(out2_hbm.at[ui_v], p2u_v, sem).wait()
    pltpu.async_copy(iemb_hbm.at[ii_v], ie_v, sem).wait()
    for g in range(PB // 16):
        ii_v[pl.ds(g * 16, 16)] = ii_v[pl.ds(g * 16, 16)] + NUM_USERS
    pltpu.async_copy(out1_hbm.at[ii_v], p1i_v, sem).wait()
    pltpu.async_copy(out2_hbm.at[ii_v], p2i_v, sem).wait()

    def _pair(e, _):
        acc = jnp.zeros((16,), jnp.float32)
        for q in range(4):
            qs = pl.ds(q * 16, 16)
            acc = acc + ue_v[e, qs] * ie_v[e, qs]
            acc = acc + p1u_v[e, qs] * p1i_v[e, qs]
            acc = acc + p2u_v[e, qs] * p2i_v[e, qs]
        tot = jnp.sum(acc)
        plsc.store_scatter(sc_v, [jnp.full((16,), e, jnp.int32)],
                           jnp.full((16,), 0.0, jnp.float32) + tot,
                           mask=iota == 0)
        return 0
    lax.fori_loop(0, PB, _pair, 0)
    pltpu.sync_copy(sc_v, scores_hbm.at[pl.ds(base, PB)])


_score = functools.partial(
    pl.kernel,
    out_type=jax.ShapeDtypeStruct((B,), jnp.float32),
    mesh=_mesh,
    scratch_types=[
        pltpu.VMEM((PB,), jnp.int32),
        pltpu.VMEM((PB,), jnp.int32),
        pltpu.VMEM((PB, D), jnp.float32),
        pltpu.VMEM((PB, D), jnp.float32),
        pltpu.VMEM((PB, D), jnp.float32),
        pltpu.VMEM((PB, D), jnp.float32),
        pltpu.VMEM((PB, D), jnp.float32),
        pltpu.VMEM((PB, D), jnp.float32),
        pltpu.VMEM((PB,), jnp.float32),
        pltpu.SemaphoreType.DMA,
    ],
    compiler_params=_sc_params,
)(_score_body)


BR = 2000


def _dense_body(x_ref, rel_ref, w1_ref, w2_ref, o_ref):
    x = x_ref[...]
    rel = rel_ref[...]
    a = jnp.dot(x + rel, w1_ref[...], preferred_element_type=jnp.float32)
    b = jnp.dot(rel * x, w2_ref[...], preferred_element_type=jnp.float32)
    o = a + b
    o_ref[...] = jnp.where(o >= 0, o, 0.2 * o)


def _dense(x, rel, w1, w2):
    return pl.pallas_call(
        _dense_body,
        grid=(N // BR,),
        in_specs=[
            pl.BlockSpec((BR, D), lambda i: (i, 0)),
            pl.BlockSpec((BR, D), lambda i: (i, 0)),
            pl.BlockSpec((D, D), lambda i: (0, 0)),
            pl.BlockSpec((D, D), lambda i: (0, 0)),
        ],
        out_specs=pl.BlockSpec((BR, D), lambda i: (i, 0)),
        out_shape=jax.ShapeDtypeStruct((N, D), jnp.float32),
    )(x, rel, w1, w2)


def kernel(user_index, item_index, edge_index, edge_vals, user_emb, item_emb,
           W1_0, W2_0, W1_1, W2_1):
    x = jnp.concatenate([user_emb, item_emb], axis=0)
    dst = edge_index[0]
    src = edge_index[1]
    rel1 = _spmm(dst, src, edge_vals, x)
    out1 = _dense(x, rel1, W1_0, W2_0)
    rel2 = _spmm(dst, src, edge_vals, out1)
    out2 = _dense(out1, rel2, W1_1, W2_1)
    return _score(user_index, item_index, user_emb, item_emb, out1, out2)
